# baseline (speedup 1.0000x reference)
import functools

import jax
import jax.numpy as jnp
from jax import lax
from jax.experimental import pallas as pl
from jax.experimental.pallas import tpu as pltpu
from jax.experimental.pallas import tpu_sc as plsc

_B, _C, _N = 64, 32, 4096
_NS = 18
_NT = _B - _NS
_L = _NS * _C * _N
_NW = 32
_PER_W = _L // _NW
_CHUNK = 4096
_UNROLL = 8


def _threefry_bits(i):
    ks0 = jnp.uint32(0)
    ks1 = jnp.uint32(42)
    ks2 = jnp.uint32(0x1BD11BDA) ^ ks0 ^ ks1
    ks = (ks0, ks1, ks2)
    rots = ((13, 15, 26, 6), (17, 29, 16, 24))

    def rotl(x, r):
        return (x << jnp.uint32(r)) | (x >> jnp.uint32(32 - r))

    x0 = i + ks1
    x1 = rotl(x0, 13) ^ x0
    for r in rots[0][1:]:
        x0 = x0 + x1
        x1 = rotl(x1, r) ^ x0
    x0 = x0 + ks[1]
    x1 = x1 + ks[2] + jnp.uint32(1)
    for grp in range(1, 5):
        for r in rots[grp % 2]:
            x0 = x0 + x1
            x1 = rotl(x1, r) ^ x0
        x0 = x0 + ks[(grp + 1) % 3]
        x1 = x1 + ks[(grp + 2) % 3] + jnp.uint32(grp + 1)
    return x0 ^ x1


def _bits_to_uniform(bits):
    fb = (bits >> jnp.uint32(9)) | jnp.uint32(0x3F800000)
    return jax.lax.bitcast_convert_type(fb, jnp.float32) - jnp.float32(1.0)


def _gumbel_from_uniform(u):
    eps = jnp.float32(1e-20)
    return -jnp.log(-jnp.log(u + eps) + eps)


def _sc_body(out_hbm, scratch):
    wid = lax.axis_index("s") * 2 + lax.axis_index("c")
    base = wid * _PER_W
    full_base = _NT * _C * _N

    def chunk_body(ch, carry):
        off = base + ch * _CHUNK

        def vec_body(k, carry2):
            for j in range(_UNROLL):
                pos = off + k * (16 * _UNROLL) + j * 16
                i = (lax.iota(jnp.uint32, 16)
                     + (full_base + pos).astype(jnp.uint32))
                u = _bits_to_uniform(_threefry_bits(i))
                scratch[pl.ds(k * (16 * _UNROLL) + j * 16, 16)] = u
            return carry2

        lax.fori_loop(0, _CHUNK // (16 * _UNROLL), vec_body, 0, unroll=False)
        pltpu.sync_copy(scratch, out_hbm.at[pl.ds(off, _CHUNK)])
        return carry

    lax.fori_loop(0, _PER_W // _CHUNK, chunk_body, 0, unroll=False)


@functools.lru_cache(maxsize=1)
def _sc_uniform_fn():
    mesh = plsc.VectorSubcoreMesh(core_axis_name="c", subcore_axis_name="s")
    return pl.kernel(
        _sc_body,
        out_type=jax.ShapeDtypeStruct((_L,), jnp.float32),
        mesh=mesh,
        scratch_types=[pltpu.VMEM((_CHUNK,), jnp.float32)],
    )


def _finish(logp, g, o_ref):
    z = logp + g
    o_ref[0] = (z[0:1, :] >= jnp.max(z, axis=0, keepdims=True)).astype(
        jnp.float32)


def _logp(l):
    m = jnp.max(l, axis=0, keepdims=True)
    e = jnp.exp(l - m)
    p = e / jnp.sum(e, axis=0, keepdims=True)
    return jnp.log(p)


def _tc_main_body(l_ref, o_ref):
    b = pl.program_id(0)
    base = (b * (_C * _N)).astype(jnp.uint32)
    row = jax.lax.broadcasted_iota(jnp.uint32, (_C, _N), 0) << jnp.uint32(12)
    col = jax.lax.broadcasted_iota(jnp.uint32, (_C, _N), 1)
    u = _bits_to_uniform(_threefry_bits(base + row + col))
    _finish(_logp(l_ref[0]), _gumbel_from_uniform(u), o_ref)


def _tc_tail_body(l_ref, u_ref, o_ref):
    _finish(_logp(l_ref[0]), _gumbel_from_uniform(u_ref[0]), o_ref)


def kernel(logits):
    u_tail = _sc_uniform_fn()().reshape(_NS, _C, _N)

    out_main = pl.pallas_call(
        _tc_main_body,
        grid=(_NT,),
        in_specs=[pl.BlockSpec((1, _C, _N), lambda b: (b, 0, 0))],
        out_specs=pl.BlockSpec((1, 1, _N), lambda b: (b, 0, 0)),
        out_shape=jax.ShapeDtypeStruct((_NT, 1, _N), jnp.float32),
        compiler_params=pltpu.CompilerParams(
            dimension_semantics=("arbitrary",),
        ),
    )(logits)

    out_tail = pl.pallas_call(
        _tc_tail_body,
        grid=(_NS,),
        in_specs=[
            pl.BlockSpec((1, _C, _N), lambda b: (b + _NT, 0, 0)),
            pl.BlockSpec((1, _C, _N), lambda b: (b, 0, 0)),
        ],
        out_specs=pl.BlockSpec((1, 1, _N), lambda b: (b, 0, 0)),
        out_shape=jax.ShapeDtypeStruct((_NS, 1, _N), jnp.float32),
        compiler_params=pltpu.CompilerParams(
            dimension_semantics=("arbitrary",),
        ),
    )(logits, u_tail)

    return jnp.concatenate([out_main, out_tail], axis=0)


# speedup vs baseline: 1.0258x; 1.0258x over previous
import functools

import jax
import jax.numpy as jnp
from jax import lax
from jax.experimental import pallas as pl
from jax.experimental.pallas import tpu as pltpu
from jax.experimental.pallas import tpu_sc as plsc

_B, _C, _N = 64, 32, 4096
_NS = 20
_NT = _B - _NS
_L = _NS * _C * _N
_NW = 32
_PER_W = _L // _NW
_CHUNK = 4096
_UNROLL = 8


def _threefry_bits(i):
    ks0 = jnp.uint32(0)
    ks1 = jnp.uint32(42)
    ks2 = jnp.uint32(0x1BD11BDA) ^ ks0 ^ ks1
    ks = (ks0, ks1, ks2)
    rots = ((13, 15, 26, 6), (17, 29, 16, 24))

    def rotl(x, r):
        return (x << jnp.uint32(r)) | (x >> jnp.uint32(32 - r))

    x0 = i + ks1
    x1 = rotl(x0, 13) ^ x0
    for r in rots[0][1:]:
        x0 = x0 + x1
        x1 = rotl(x1, r) ^ x0
    x0 = x0 + ks[1]
    x1 = x1 + ks[2] + jnp.uint32(1)
    for grp in range(1, 5):
        for r in rots[grp % 2]:
            x0 = x0 + x1
            x1 = rotl(x1, r) ^ x0
        x0 = x0 + ks[(grp + 1) % 3]
        x1 = x1 + ks[(grp + 2) % 3] + jnp.uint32(grp + 1)
    return x0 ^ x1


def _bits_to_uniform(bits):
    fb = (bits >> jnp.uint32(9)) | jnp.uint32(0x3F800000)
    return jax.lax.bitcast_convert_type(fb, jnp.float32) - jnp.float32(1.0)


def _gumbel_from_uniform(u):
    eps = jnp.float32(1e-20)
    return -jnp.log(-jnp.log(u + eps) + eps)


def _sc_body(out_hbm, scratch):
    wid = lax.axis_index("s") * 2 + lax.axis_index("c")
    base = wid * _PER_W
    full_base = _NT * _C * _N

    def chunk_body(ch, carry):
        off = base + ch * _CHUNK

        def vec_body(k, carry2):
            for j in range(_UNROLL):
                pos = off + k * (16 * _UNROLL) + j * 16
                i = (lax.iota(jnp.uint32, 16)
                     + (full_base + pos).astype(jnp.uint32))
                u = _bits_to_uniform(_threefry_bits(i))
                scratch[pl.ds(k * (16 * _UNROLL) + j * 16, 16)] = u
            return carry2

        lax.fori_loop(0, _CHUNK // (16 * _UNROLL), vec_body, 0, unroll=False)
        pltpu.sync_copy(scratch, out_hbm.at[pl.ds(off, _CHUNK)])
        return carry

    lax.fori_loop(0, _PER_W // _CHUNK, chunk_body, 0, unroll=False)


@functools.lru_cache(maxsize=1)
def _sc_uniform_fn():
    mesh = plsc.VectorSubcoreMesh(core_axis_name="c", subcore_axis_name="s")
    return pl.kernel(
        _sc_body,
        out_type=jax.ShapeDtypeStruct((_L,), jnp.float32),
        mesh=mesh,
        scratch_types=[pltpu.VMEM((_CHUNK,), jnp.float32)],
    )


def _finish(logp, g, o_ref):
    z = logp + g
    o_ref[0] = (z[0:1, :] >= jnp.max(z, axis=0, keepdims=True)).astype(
        jnp.float32)


def _logp(l):
    m = jnp.max(l, axis=0, keepdims=True)
    e = jnp.exp(l - m)
    p = e / jnp.sum(e, axis=0, keepdims=True)
    return jnp.log(p)


def _tc_main_body(l_ref, o_ref):
    b = pl.program_id(0)
    base = (b * (_C * _N)).astype(jnp.uint32)
    row = jax.lax.broadcasted_iota(jnp.uint32, (_C, _N), 0) << jnp.uint32(12)
    col = jax.lax.broadcasted_iota(jnp.uint32, (_C, _N), 1)
    u = _bits_to_uniform(_threefry_bits(base + row + col))
    _finish(_logp(l_ref[0]), _gumbel_from_uniform(u), o_ref)


def _tc_tail_body(l_ref, u_ref, o_ref):
    _finish(_logp(l_ref[0]), _gumbel_from_uniform(u_ref[0]), o_ref)


def kernel(logits):
    u_tail = _sc_uniform_fn()().reshape(_NS, _C, _N)

    out_main = pl.pallas_call(
        _tc_main_body,
        grid=(_NT,),
        in_specs=[pl.BlockSpec((1, _C, _N), lambda b: (b, 0, 0))],
        out_specs=pl.BlockSpec((1, 1, _N), lambda b: (b, 0, 0)),
        out_shape=jax.ShapeDtypeStruct((_NT, 1, _N), jnp.float32),
        compiler_params=pltpu.CompilerParams(
            dimension_semantics=("arbitrary",),
        ),
    )(logits)

    out_tail = pl.pallas_call(
        _tc_tail_body,
        grid=(_NS,),
        in_specs=[
            pl.BlockSpec((1, _C, _N), lambda b: (b + _NT, 0, 0)),
            pl.BlockSpec((1, _C, _N), lambda b: (b, 0, 0)),
        ],
        out_specs=pl.BlockSpec((1, 1, _N), lambda b: (b, 0, 0)),
        out_shape=jax.ShapeDtypeStruct((_NS, 1, _N), jnp.float32),
        compiler_params=pltpu.CompilerParams(
            dimension_semantics=("arbitrary",),
        ),
    )(logits, u_tail)

    return jnp.concatenate([out_main, out_tail], axis=0)
